# Initial kernel scaffold; baseline (speedup 1.0000x reference)
import jax
import jax.numpy as jnp
from jax.experimental import pallas as pl


def kernel(x, edge_index, edge_weight, Wx1, Wh1, bx1, bh1, Wx2, Wh2, bx2, bh2, Wx3, Wh3, bx3, bh3, Wlin, blin):
    raise NotImplementedError("write your pallas kernel here")


# speedup vs baseline: 8.1368x; 8.1368x over previous
import functools

import jax
import jax.numpy as jnp
from jax import lax
from jax.experimental import pallas as pl
from jax.experimental.pallas import tpu as pltpu
from jax.experimental.pallas import tpu_sc as plsc

N = 10000
E = 320000
NC = 2
NS = 16
NPAD = 10240
CE = 2000
B = 80
ET = E // NS
NBATCH = ET // B
EWK = E // (NC * NS)
RT = NPAD // NS
RC = 128
KCH = 5


def _rsqrt16(x):
    bits = lax.bitcast_convert_type(x, jnp.int32)
    y = lax.bitcast_convert_type(
        jnp.int32(0x5F3759DF) - lax.shift_right_logical(bits, 1), jnp.float32)
    for _ in range(4):
        y = y * (1.5 - 0.5 * x * y * y)
    return jnp.where(x > 0.0, y, 0.0)


def _make_pre():
    mesh = plsc.VectorSubcoreMesh(core_axis_name="c", subcore_axis_name="s")

    @functools.partial(
        pl.kernel,
        out_type=jax.ShapeDtypeStruct((E,), jnp.float32),
        mesh=mesh,
        compiler_params=pltpu.CompilerParams(needs_layout_passes=False),
        scratch_types=[
            pltpu.VMEM((NPAD,), jnp.float32),
            pltpu.VMEM((NPAD,), jnp.float32),
            pltpu.VMEM_SHARED((NPAD,), jnp.float32),
            pltpu.VMEM((B,), jnp.int32),
            pltpu.VMEM((B,), jnp.float32),
            pltpu.VMEM((CE,), jnp.int32),
            pltpu.VMEM((CE,), jnp.int32),
            pltpu.VMEM((CE,), jnp.float32),
            pltpu.VMEM((CE,), jnp.float32),
        ],
    )
    def pre(row_h, col_h, ew_h, lw_h, deg_v, dis_v, deg_s,
            ri80_v, w80_v, ri_v, ci_v, w_v, lwo_v):
        c = lax.axis_index("c")
        s = lax.axis_index("s")

        def zero_deg(i, _):
            deg_v[pl.ds(i * 16, 16)] = jnp.zeros((16,), jnp.float32)
            return 0
        lax.fori_loop(0, NPAD // 16, zero_deg, 0)

        ztile = NPAD // NS
        pltpu.sync_copy(deg_v.at[pl.ds(0, ztile)],
                        deg_s.at[pl.ds(s * ztile, ztile)])
        plsc.subcore_barrier()

        def deg_batch(i, _):
            base = s * ET + i * B
            pltpu.sync_copy(row_h.at[pl.ds(base, B)], ri80_v)
            pltpu.sync_copy(ew_h.at[pl.ds(base, B)], w80_v)
            pltpu.sync_copy(w80_v, deg_s.at[ri80_v], add=True)
            return 0
        lax.fori_loop(0, NBATCH, deg_batch, 0)
        plsc.subcore_barrier()
        pltpu.sync_copy(deg_s, deg_v)

        def mkdis(i, _):
            sl = pl.ds(i * 16, 16)
            dis_v[sl] = _rsqrt16(deg_v[sl])
            return 0
        lax.fori_loop(0, NPAD // 16, mkdis, 0)

        wid = c * NS + s

        def lw_chunk(ch, _):
            base = wid * EWK + ch * CE
            pltpu.sync_copy(row_h.at[pl.ds(base, CE)], ri_v)
            pltpu.sync_copy(col_h.at[pl.ds(base, CE)], ci_v)
            pltpu.sync_copy(ew_h.at[pl.ds(base, CE)], w_v)

            def inner(q, _):
                sl = pl.ds(q * 16, 16)
                rv = ri_v[sl]
                cv = ci_v[sl]
                wv = w_v[sl]
                dr = plsc.load_gather(dis_v, [rv])
                dc = plsc.load_gather(dis_v, [cv])
                val = -(dr * wv * dc)
                lwo_v[sl] = jnp.where(rv == cv, val - 1.0, val)
                return 0
            lax.fori_loop(0, CE // 16, inner, 0)
            pltpu.sync_copy(lwo_v, lw_h.at[pl.ds(base, CE)])
            return 0
        lax.fori_loop(0, EWK // CE, lw_chunk, 0)

    return pre


def _make_prop(Ch):
    mesh = plsc.VectorSubcoreMesh(core_axis_name="c", subcore_axis_name="s")
    tsd = jax.ShapeDtypeStruct((2 * NPAD, Ch), jnp.float32)

    @functools.partial(
        pl.kernel,
        out_type=[tsd, tsd, tsd, tsd],
        mesh=mesh,
        compiler_params=pltpu.CompilerParams(needs_layout_passes=False,
                                             use_tc_tiling_on_sc=False),
        scratch_types=[
            pltpu.VMEM_SHARED((NPAD, Ch), jnp.float32),
            pltpu.VMEM((B,), jnp.int32),
            pltpu.VMEM((B,), jnp.int32),
            pltpu.VMEM((B,), jnp.float32),
            pltpu.VMEM((B, Ch), jnp.float32),
            pltpu.VMEM((RC, Ch), jnp.float32),
            pltpu.VMEM((RC, Ch), jnp.float32),
            pltpu.SemaphoreType.DMA,
        ],
    )
    def prop(t0_h, row_h, col_h, lw_h, t1_h, t2_h, t3_h, t4_h,
             acc_s, srcb_v, dstb_v, lwb_v, rows_v, p_v, tb_v, sem):
        c = lax.axis_index("c")
        s = lax.axis_index("s")
        ebase = s * ET
        rbase = s * RT
        coff = c * NPAD

        def zrow(r, _):
            for v in range(Ch // 16):
                tb_v[r, pl.ds(v * 16, 16)] = jnp.zeros((16,), jnp.float32)
            return 0
        lax.fori_loop(0, RC, zrow, 0)

        ins = [t0_h, t1_h, t2_h, t3_h]
        prevs = [None, t0_h, t1_h, t2_h]
        outs = [t1_h, t2_h, t3_h, t4_h]
        for k in range(4):
            tin, tprev, tout = ins[k], prevs[k], outs[k]

            for j in range(RT // RC):
                pltpu.sync_copy(tb_v, acc_s.at[pl.ds(rbase + j * RC, RC)])
            plsc.subcore_barrier()

            def batch(i, _):
                eb = ebase + i * B
                pltpu.sync_copy(row_h.at[pl.ds(eb, B)], srcb_v)

                def adj(q, _):
                    sl = pl.ds(q * 16, 16)
                    srcb_v[sl] = srcb_v[sl] + coff
                    return 0
                lax.fori_loop(0, B // 16, adj, 0)
                pltpu.async_copy(tin.at[srcb_v], rows_v, sem).wait()
                pltpu.sync_copy(lw_h.at[pl.ds(eb, B)], lwb_v)

                def scale(e, _):
                    esplat = jnp.zeros((16,), jnp.int32) + e
                    w = plsc.load_gather(lwb_v, [esplat])
                    for v in range(Ch // 16):
                        sl = pl.ds(v * 16, 16)
                        rows_v[e, sl] = rows_v[e, sl] * w
                    return 0
                lax.fori_loop(0, B, scale, 0)
                pltpu.sync_copy(col_h.at[pl.ds(eb, B)], dstb_v)
                pltpu.sync_copy(rows_v, acc_s.at[dstb_v], add=True)
                return 0
            lax.fori_loop(0, NBATCH, batch, 0)
            plsc.subcore_barrier()

            for j in range(RT // RC):
                r0 = rbase + j * RC
                pltpu.sync_copy(acc_s.at[pl.ds(r0, RC)], p_v)
                if k == 0:
                    pltpu.sync_copy(p_v, tout.at[pl.ds(coff + r0, RC)])
                else:
                    pltpu.sync_copy(tprev.at[pl.ds(coff + r0, RC)], tb_v)

                    def comb(r, _):
                        for v in range(Ch // 16):
                            sl = pl.ds(v * 16, 16)
                            p_v[r, sl] = 2.0 * p_v[r, sl] - tb_v[r, sl]
                        return 0
                    lax.fori_loop(0, RC, comb, 0)
                    pltpu.sync_copy(p_v, tout.at[pl.ds(coff + r0, RC)])
            if k == 3:
                break
            plsc.subcore_barrier()
            if k >= 1:
                lax.fori_loop(0, RC, zrow, 0)

    return prop


def _mm_body(nt, Hout, last, *refs):
    t_refs = refs[:KCH]
    w_ref, b_ref = refs[KCH], refs[KCH + 1]
    if last:
        wlin_ref, blin_ref = refs[KCH + 2], refs[KCH + 3]
        out_ref = refs[KCH + 4]
    else:
        out_ref = refs[KCH + 2]
    bn = nt
    acc = jnp.zeros((bn, 2 * Hout), jnp.float32)
    for k in range(KCH):
        for h in range(2):
            acc = acc + jnp.dot(t_refs[k][h], w_ref[k, h],
                                preferred_element_type=jnp.float32)
    b = b_ref[0]
    z = jax.nn.sigmoid(acc[:, :Hout] + b[:Hout])
    t = jnp.tanh(acc[:, Hout:] + b[Hout:])
    hval = jax.nn.relu((1.0 - z) * t)
    if last:
        y = jnp.dot(hval, wlin_ref[...], preferred_element_type=jnp.float32)
        out_ref[...] = y + blin_ref[0, 0]
    else:
        half = Hout // 2
        out_ref[0] = hval[:, :half]
        out_ref[1] = hval[:, half:]


def _run_mm(ts, wc, bc, Cin, Hout, last=False, wlin=None, blin=None):
    Ch = Cin // 2
    bn = 1024
    grid = (NPAD // bn,)
    t_spec = pl.BlockSpec((2, bn, Ch), lambda i: (0, i, 0))
    in_specs = [t_spec] * KCH + [
        pl.BlockSpec((KCH, 2, Ch, 2 * Hout), lambda i: (0, 0, 0, 0)),
        pl.BlockSpec((1, 2 * Hout), lambda i: (0, 0)),
    ]
    args = list(ts) + [wc, bc]
    if last:
        in_specs += [
            pl.BlockSpec((Hout, 1), lambda i: (0, 0)),
            pl.BlockSpec((1, 1), lambda i: (0, 0)),
        ]
        args += [wlin, blin]
        out_spec = pl.BlockSpec((bn, 1), lambda i: (i, 0))
        out_shape = jax.ShapeDtypeStruct((NPAD, 1), jnp.float32)
    else:
        out_spec = pl.BlockSpec((2, bn, Hout // 2), lambda i: (0, i, 0))
        out_shape = jax.ShapeDtypeStruct((2, NPAD, Hout // 2), jnp.float32)
    fn = pl.pallas_call(
        functools.partial(_mm_body, bn, Hout, last),
        grid=grid,
        in_specs=in_specs,
        out_specs=out_spec,
        out_shape=out_shape,
    )
    return fn(*args)


def _mkw(Wx, bx, bh):
    K, ic, oc = Wx.shape[1], Wx.shape[2], Wx.shape[3]
    wc = jnp.concatenate([Wx[0], Wx[2]], axis=-1)
    wc = wc.reshape(K, 2, ic // 2, 2 * oc)
    bc = jnp.concatenate([bx[0] + bh[0], bx[2] + bh[2]])
    return wc, bc.reshape(1, 2 * oc)


def kernel(x, edge_index, edge_weight, Wx1, Wh1, bx1, bh1, Wx2, Wh2, bx2, bh2,
           Wx3, Wh3, bx3, bh3, Wlin, blin):
    row = edge_index[0]
    col = edge_index[1]
    pre = _make_pre()
    lw = pre(row, col, edge_weight)

    prop64 = _make_prop(64)
    prop128 = _make_prop(128)

    def layer(xin2, Wx, bx, bh, Cin, Hout, last=False):
        propfn = prop64 if Cin == 128 else prop128
        t1, t2, t3, t4 = propfn(xin2, row, col, lw)
        wc, bc = _mkw(Wx, bx, bh)
        Ch = Cin // 2
        ts = [a.reshape(2, NPAD, Ch) for a in (xin2, t1, t2, t3, t4)]
        if last:
            return _run_mm(ts, wc, bc, Cin, Hout, last=True,
                           wlin=Wlin, blin=blin.reshape(1, 1))[:N, 0]
        h = _run_mm(ts, wc, bc, Cin, Hout)
        return h.reshape(2 * NPAD, Hout // 2)

    pad = ((0, NPAD - N), (0, 0))
    x2 = jnp.concatenate(
        [jnp.pad(x[:, :64], pad), jnp.pad(x[:, 64:], pad)], axis=0)
    h1 = layer(x2, Wx1, bx1, bh1, 128, 256)
    h2 = layer(h1, Wx2, bx2, bh2, 256, 128)
    y = layer(h2, Wx3, bx3, bh3, 128, 64, last=True)
    return y


# speedup vs baseline: 32.3105x; 3.9709x over previous
import functools

import jax
import jax.numpy as jnp
from jax import lax
from jax.experimental import pallas as pl
from jax.experimental.pallas import tpu as pltpu
from jax.experimental.pallas import tpu_sc as plsc

N = 10000
E = 320000
NC = 2
NS = 16
NPAD = 10240
CE = 2000
B = 80
ET = E // NS
NBATCH = ET // B
EWK = E // (NC * NS)
RT = NPAD // NS
RC = 128
KCH = 5
CH = 64
BB = 128
NB2 = 156
ETM = NB2 * BB
TBASE = NS * ETM
TAIL = (E - TBASE) // NS


def _rsqrt16(x):
    bits = lax.bitcast_convert_type(x, jnp.int32)
    y = lax.bitcast_convert_type(
        jnp.int32(0x5F3759DF) - lax.shift_right_logical(bits, 1), jnp.float32)
    for _ in range(4):
        y = y * (1.5 - 0.5 * x * y * y)
    return jnp.where(x > 0.0, y, 0.0)


def _make_pre():
    mesh = plsc.VectorSubcoreMesh(core_axis_name="c", subcore_axis_name="s")

    @functools.partial(
        pl.kernel,
        out_type=jax.ShapeDtypeStruct((E,), jnp.float32),
        mesh=mesh,
        compiler_params=pltpu.CompilerParams(needs_layout_passes=False),
        scratch_types=[
            pltpu.VMEM((NPAD,), jnp.float32),
            pltpu.VMEM((NPAD,), jnp.float32),
            pltpu.VMEM_SHARED((NPAD,), jnp.float32),
            pltpu.VMEM((B,), jnp.int32),
            pltpu.VMEM((B,), jnp.int32),
            pltpu.VMEM((ET,), jnp.float32),
            pltpu.VMEM((CE,), jnp.int32),
            pltpu.VMEM((CE,), jnp.int32),
            pltpu.VMEM((CE,), jnp.float32),
            pltpu.VMEM((CE,), jnp.float32),
            pltpu.SemaphoreType.DMA,
            pltpu.SemaphoreType.DMA,
            pltpu.SemaphoreType.DMA,
            pltpu.SemaphoreType.DMA,
        ],
    )
    def pre(row_h, col_h, ew_h, lw_h, deg_v, dis_v, deg_s,
            ri0_v, ri1_v, wall_v, ri_v, ci_v, w_v, lwo_v,
            i0, i1, s0, s1):
        c = lax.axis_index("c")
        s = lax.axis_index("s")

        def zero_deg(i, _):
            deg_v[pl.ds(i * 16, 16)] = jnp.zeros((16,), jnp.float32)
            return 0
        lax.fori_loop(0, NPAD // 16, zero_deg, 0)

        ztile = NPAD // NS
        pltpu.sync_copy(deg_v.at[pl.ds(0, ztile)],
                        deg_s.at[pl.ds(s * ztile, ztile)])
        plsc.subcore_barrier()

        pltpu.sync_copy(ew_h.at[pl.ds(s * ET, ET)], wall_v)
        ri_b = (ri0_v, ri1_v)
        isems = (i0, i1)
        ssems = (s0, s1)
        pltpu.async_copy(row_h.at[pl.ds(s * ET, B)], ri_b[0], isems[0])

        def deg_pair(i2, _):
            for b in range(2):
                bnx = (b + 1) % 2
                i = i2 * 2 + b

                @pl.when(i >= 1)
                def _():
                    pltpu.make_async_copy(
                        wall_v.at[pl.ds(i * B, B)],
                        deg_s.at[ri_b[bnx]], ssems[bnx]).wait()

                @pl.when(i + 1 < NBATCH)
                def _():
                    pltpu.async_copy(
                        row_h.at[pl.ds(s * ET + (i + 1) * B, B)],
                        ri_b[bnx], isems[bnx])

                pltpu.make_async_copy(
                    row_h.at[pl.ds(s * ET + i * B, B)],
                    ri_b[b], isems[b]).wait()
                pltpu.async_copy(wall_v.at[pl.ds(i * B, B)],
                                 deg_s.at[ri_b[b]], ssems[b], add=True)
            return 0
        lax.fori_loop(0, NBATCH // 2, deg_pair, 0)
        pltpu.make_async_copy(wall_v.at[pl.ds(0, B)],
                              deg_s.at[ri_b[1]], ssems[1]).wait()
        plsc.subcore_barrier()
        pltpu.sync_copy(deg_s, deg_v)

        def mkdis(i, _):
            sl = pl.ds(i * 16, 16)
            dis_v[sl] = _rsqrt16(deg_v[sl])
            return 0
        lax.fori_loop(0, NPAD // 16, mkdis, 0)

        wid = c * NS + s

        def lw_chunk(ch, _):
            base = wid * EWK + ch * CE
            pltpu.sync_copy(row_h.at[pl.ds(base, CE)], ri_v)
            pltpu.sync_copy(col_h.at[pl.ds(base, CE)], ci_v)
            pltpu.sync_copy(ew_h.at[pl.ds(base, CE)], w_v)

            def inner(q, _):
                sl = pl.ds(q * 16, 16)
                rv = ri_v[sl]
                cv = ci_v[sl]
                wv = w_v[sl]
                dr = plsc.load_gather(dis_v, [rv])
                dc = plsc.load_gather(dis_v, [cv])
                val = -(dr * wv * dc)
                lwo_v[sl] = jnp.where(rv == cv, val - 1.0, val)
                return 0
            lax.fori_loop(0, CE // 16, inner, 0)
            pltpu.sync_copy(lwo_v, lw_h.at[pl.ds(base, CE)])
            return 0
        lax.fori_loop(0, EWK // CE, lw_chunk, 0)

    return pre


def _make_prop(Q):
    mesh = plsc.VectorSubcoreMesh(core_axis_name="c", subcore_axis_name="s")
    tsd = jax.ShapeDtypeStruct((Q * NPAD, CH), jnp.float32)

    @functools.partial(
        pl.kernel,
        out_type=[tsd, tsd, tsd, tsd],
        mesh=mesh,
        compiler_params=pltpu.CompilerParams(needs_layout_passes=False,
                                             use_tc_tiling_on_sc=False),
        scratch_types=[
            pltpu.VMEM_SHARED((NPAD, CH), jnp.float32),
            pltpu.VMEM((ETM,), jnp.int32),
            pltpu.VMEM((ETM,), jnp.float32),
            pltpu.VMEM((BB,), jnp.int32),
            pltpu.VMEM((BB,), jnp.int32),
            pltpu.VMEM((BB,), jnp.int32),
            pltpu.VMEM((BB, CH), jnp.float32),
            pltpu.VMEM((BB, CH), jnp.float32),
            pltpu.VMEM((BB, CH), jnp.float32),
            pltpu.VMEM((RC, CH), jnp.float32),
            pltpu.VMEM((RC, CH), jnp.float32),
            pltpu.VMEM((TAIL,), jnp.int32),
            pltpu.VMEM((TAIL,), jnp.int32),
            pltpu.VMEM((TAIL,), jnp.float32),
            pltpu.VMEM((TAIL, CH), jnp.float32),
            pltpu.SemaphoreType.DMA,
            pltpu.SemaphoreType.DMA,
            pltpu.SemaphoreType.DMA,
            pltpu.SemaphoreType.DMA,
            pltpu.SemaphoreType.DMA,
            pltpu.SemaphoreType.DMA,
            pltpu.SemaphoreType.DMA,
            pltpu.SemaphoreType.DMA,
            pltpu.SemaphoreType.DMA,
            pltpu.SemaphoreType.DMA,
        ],
    )
    def prop(t0_h, row_h, col_h, lw_h, t1_h, t2_h, t3_h, t4_h,
             acc_s, srca_v, lwr_v, d0_v, d1_v, d2_v, r0_v, r1_v, r2_v,
             p_v, tb_v, srct_v, dstt_v, lwt_v, rowst_v,
             g0, g1, g2, s0, s1, s2, e0, e1, e2, st):
        rows_b = (r0_v, r1_v, r2_v)
        dst_b = (d0_v, d1_v, d2_v)
        gsems = (g0, g1, g2)
        ssems = (s0, s1, s2)
        dsems = (e0, e1, e2)
        c = lax.axis_index("c")
        s = lax.axis_index("s")
        rbase = s * RT
        mbase = s * ETM
        tbase = TBASE + s * TAIL
        QC = Q // 2

        pltpu.sync_copy(row_h.at[pl.ds(mbase, ETM)], srca_v)
        pltpu.sync_copy(lw_h.at[pl.ds(mbase, ETM)], lwr_v)
        pltpu.sync_copy(row_h.at[pl.ds(tbase, TAIL)], srct_v)
        pltpu.sync_copy(lw_h.at[pl.ds(tbase, TAIL)], lwt_v)
        pltpu.sync_copy(col_h.at[pl.ds(tbase, TAIL)], dstt_v)

        def zrow(r, _):
            for v in range(CH // 16):
                tb_v[r, pl.ds(v * 16, 16)] = jnp.zeros((16,), jnp.float32)
            return 0
        lax.fori_loop(0, RC, zrow, 0)

        ins = [t0_h, t1_h, t2_h, t3_h]
        prevs = [None, t0_h, t1_h, t2_h]
        outs = [t1_h, t2_h, t3_h, t4_h]
        for q in range(QC):
            coff = (c * QC + q) * NPAD
            delta = coff if q == 0 else NPAD

            def adj(i, _):
                sl = pl.ds(i * 16, 16)
                srca_v[sl] = srca_v[sl] + delta
                return 0
            lax.fori_loop(0, ETM // 16, adj, 0, unroll=8)

            def adjt(i, _):
                sl = pl.ds(i * 16, 16)
                srct_v[sl] = srct_v[sl] + delta
                return 0
            lax.fori_loop(0, TAIL // 16, adjt, 0)

            for k in range(4):
                tin, tprev, tout = ins[k], prevs[k], outs[k]

                for j in range(RT // RC):
                    pltpu.sync_copy(tb_v, acc_s.at[pl.ds(rbase + j * RC, RC)])
                plsc.subcore_barrier()

                pltpu.async_copy(tin.at[srca_v.at[pl.ds(0, BB)]],
                                 rows_b[0], gsems[0])
                pltpu.async_copy(col_h.at[pl.ds(mbase, BB)],
                                 dst_b[0], dsems[0])

                def triple(i3, _):
                    for b in range(3):
                        bnx = (b + 1) % 3
                        i = i3 * 3 + b

                        @pl.when(i >= 2)
                        def _():
                            pltpu.make_async_copy(
                                rows_b[bnx], acc_s.at[dst_b[bnx]],
                                ssems[bnx]).wait()

                        @pl.when(i + 1 < NB2)
                        def _():
                            pltpu.async_copy(
                                tin.at[srca_v.at[pl.ds((i + 1) * BB, BB)]],
                                rows_b[bnx], gsems[bnx])
                            pltpu.async_copy(
                                col_h.at[pl.ds(mbase + (i + 1) * BB, BB)],
                                dst_b[bnx], dsems[bnx])

                        pltpu.make_async_copy(
                            tin.at[srca_v.at[pl.ds(i * BB, BB)]],
                            rows_b[b], gsems[b]).wait()
                        rv = rows_b[b]
                        base_i = i * BB

                        @plsc.parallel_loop(0, BB, unroll=8)
                        def _(e):
                            ev = jnp.zeros((16,), jnp.int32) + (base_i + e)
                            w = plsc.load_gather(lwr_v, [ev])
                            for v in range(CH // 16):
                                sl = pl.ds(v * 16, 16)
                                rv[e, sl] = rv[e, sl] * w

                        pltpu.make_async_copy(
                            col_h.at[pl.ds(mbase + i * BB, BB)],
                            dst_b[b], dsems[b]).wait()
                        pltpu.async_copy(rv, acc_s.at[dst_b[b]],
                                         ssems[b], add=True)
                    return 0
                lax.fori_loop(0, NB2 // 3, triple, 0)
                pltpu.make_async_copy(rows_b[1], acc_s.at[dst_b[1]],
                                      ssems[1]).wait()
                pltpu.make_async_copy(rows_b[2], acc_s.at[dst_b[2]],
                                      ssems[2]).wait()

                pltpu.async_copy(tin.at[srct_v], rowst_v, st).wait()

                def tscale(e, ev):
                    w = plsc.load_gather(lwt_v, [ev])
                    for v in range(CH // 16):
                        sl = pl.ds(v * 16, 16)
                        rowst_v[e, sl] = rowst_v[e, sl] * w
                    return ev + 1
                lax.fori_loop(0, TAIL, tscale, jnp.zeros((16,), jnp.int32),
                              unroll=4)
                pltpu.sync_copy(rowst_v, acc_s.at[dstt_v], add=True)
                plsc.subcore_barrier()

                for j in range(RT // RC):
                    r0 = rbase + j * RC
                    pltpu.sync_copy(acc_s.at[pl.ds(r0, RC)], p_v)
                    if k == 0:
                        pltpu.sync_copy(p_v, tout.at[pl.ds(coff + r0, RC)])
                    else:
                        pltpu.sync_copy(tprev.at[pl.ds(coff + r0, RC)], tb_v)

                        def comb(r, _):
                            for v in range(CH // 16):
                                sl = pl.ds(v * 16, 16)
                                p_v[r, sl] = 2.0 * p_v[r, sl] - tb_v[r, sl]
                            return 0
                        lax.fori_loop(0, RC, comb, 0, unroll=2)
                        pltpu.sync_copy(p_v, tout.at[pl.ds(coff + r0, RC)])
                if q == QC - 1 and k == 3:
                    break
                plsc.subcore_barrier()
                if k >= 1:
                    lax.fori_loop(0, RC, zrow, 0)

    return prop


def _mm_body(bn, Qin, Hout, last, *refs):
    t_refs = refs[:KCH]
    w_ref, b_ref = refs[KCH], refs[KCH + 1]
    if last:
        wlin_ref, blin_ref = refs[KCH + 2], refs[KCH + 3]
        out_ref = refs[KCH + 4]
    else:
        out_ref = refs[KCH + 2]
    acc = jnp.zeros((bn, 2 * Hout), jnp.float32)
    for k in range(KCH):
        for h in range(Qin):
            acc = acc + jnp.dot(t_refs[k][h], w_ref[k, h],
                                preferred_element_type=jnp.float32)
    b = b_ref[0]
    z = jax.nn.sigmoid(acc[:, :Hout] + b[:Hout])
    t = jnp.tanh(acc[:, Hout:] + b[Hout:])
    hval = jax.nn.relu((1.0 - z) * t)
    if last:
        y = jnp.dot(hval, wlin_ref[...], preferred_element_type=jnp.float32)
        out_ref[...] = y + blin_ref[0, 0]
    else:
        Qout = out_ref.shape[0]
        w = Hout // Qout
        for qo in range(Qout):
            out_ref[qo] = hval[:, qo * w:(qo + 1) * w]


def _run_mm(ts, wc, bc, Qin, Hout, Qout=2, last=False, wlin=None, blin=None):
    bn = 1024
    grid = (NPAD // bn,)
    t_spec = pl.BlockSpec((Qin, bn, CH), lambda i: (0, i, 0))
    in_specs = [t_spec] * KCH + [
        pl.BlockSpec((KCH, Qin, CH, 2 * Hout), lambda i: (0, 0, 0, 0)),
        pl.BlockSpec((1, 2 * Hout), lambda i: (0, 0)),
    ]
    args = list(ts) + [wc, bc]
    if last:
        in_specs += [
            pl.BlockSpec((Hout, 1), lambda i: (0, 0)),
            pl.BlockSpec((1, 1), lambda i: (0, 0)),
        ]
        args += [wlin, blin]
        out_spec = pl.BlockSpec((bn, 1), lambda i: (i, 0))
        out_shape = jax.ShapeDtypeStruct((NPAD, 1), jnp.float32)
    else:
        out_spec = pl.BlockSpec((Qout, bn, Hout // Qout), lambda i: (0, i, 0))
        out_shape = jax.ShapeDtypeStruct((Qout, NPAD, Hout // Qout),
                                         jnp.float32)
    fn = pl.pallas_call(
        functools.partial(_mm_body, bn, Qin, Hout, last),
        grid=grid,
        in_specs=in_specs,
        out_specs=out_spec,
        out_shape=out_shape,
    )
    return fn(*args)


def _mkw(Wx, bx, bh, Qin):
    K, ic, oc = Wx.shape[1], Wx.shape[2], Wx.shape[3]
    wc = jnp.concatenate([Wx[0], Wx[2]], axis=-1)
    wc = wc.reshape(K, Qin, ic // Qin, 2 * oc)
    bc = jnp.concatenate([bx[0] + bh[0], bx[2] + bh[2]])
    return wc, bc.reshape(1, 2 * oc)


def kernel(x, edge_index, edge_weight, Wx1, Wh1, bx1, bh1, Wx2, Wh2, bx2, bh2,
           Wx3, Wh3, bx3, bh3, Wlin, blin):
    row = edge_index[0]
    col = edge_index[1]
    pre = _make_pre()
    lw = pre(row, col, edge_weight)

    prop2 = _make_prop(2)
    prop4 = _make_prop(4)

    def layer(xinq, Qin, Wx, bx, bh, Hout, Qout, last=False):
        propfn = prop2 if Qin == 2 else prop4
        t1, t2, t3, t4 = propfn(xinq, row, col, lw)
        wc, bc = _mkw(Wx, bx, bh, Qin)
        ts = [a.reshape(Qin, NPAD, CH) for a in (xinq, t1, t2, t3, t4)]
        if last:
            return _run_mm(ts, wc, bc, Qin, Hout, last=True,
                           wlin=Wlin, blin=blin.reshape(1, 1))[:N, 0]
        h = _run_mm(ts, wc, bc, Qin, Hout, Qout=Qout)
        return h.reshape(Qout * NPAD, CH)

    pad = ((0, NPAD - N), (0, 0))
    x2 = jnp.concatenate(
        [jnp.pad(x[:, :64], pad), jnp.pad(x[:, 64:], pad)], axis=0)
    h1 = layer(x2, 2, Wx1, bx1, bh1, 256, 4)
    h2 = layer(h1, 4, Wx2, bx2, bh2, 128, 2)
    y = layer(h2, 2, Wx3, bx3, bh3, 64, 1, last=True)
    return y


# speedup vs baseline: 34.9877x; 1.0829x over previous
import functools

import jax
import jax.numpy as jnp
from jax import lax
from jax.experimental import pallas as pl
from jax.experimental.pallas import tpu as pltpu
from jax.experimental.pallas import tpu_sc as plsc

N = 10000
E = 320000
NC = 2
NS = 16
NPAD = 10240
CE = 2000
B = 80
ET = E // NS
NBATCH = ET // B
EWK = E // (NC * NS)
RT = NPAD // NS
RC = 128
KCH = 5
CH = 64
BB = 128
NB2 = 156
ETM = NB2 * BB
TBASE = NS * ETM
TAIL = (E - TBASE) // NS


def _rsqrt16(x):
    bits = lax.bitcast_convert_type(x, jnp.int32)
    y = lax.bitcast_convert_type(
        jnp.int32(0x5F3759DF) - lax.shift_right_logical(bits, 1), jnp.float32)
    for _ in range(4):
        y = y * (1.5 - 0.5 * x * y * y)
    return jnp.where(x > 0.0, y, 0.0)


def _make_pre():
    mesh = plsc.VectorSubcoreMesh(core_axis_name="c", subcore_axis_name="s")

    @functools.partial(
        pl.kernel,
        out_type=jax.ShapeDtypeStruct((E,), jnp.float32),
        mesh=mesh,
        compiler_params=pltpu.CompilerParams(needs_layout_passes=False),
        scratch_types=[
            pltpu.VMEM((NPAD,), jnp.float32),
            pltpu.VMEM((NPAD,), jnp.float32),
            pltpu.VMEM_SHARED((NPAD,), jnp.float32),
            pltpu.VMEM((B,), jnp.int32),
            pltpu.VMEM((B,), jnp.int32),
            pltpu.VMEM((ET,), jnp.float32),
            pltpu.VMEM((CE,), jnp.int32),
            pltpu.VMEM((CE,), jnp.int32),
            pltpu.VMEM((CE,), jnp.float32),
            pltpu.VMEM((CE,), jnp.float32),
            pltpu.SemaphoreType.DMA,
            pltpu.SemaphoreType.DMA,
            pltpu.SemaphoreType.DMA,
            pltpu.SemaphoreType.DMA,
        ],
    )
    def pre(row_h, col_h, ew_h, lw_h, deg_v, dis_v, deg_s,
            ri0_v, ri1_v, wall_v, ri_v, ci_v, w_v, lwo_v,
            i0, i1, s0, s1):
        c = lax.axis_index("c")
        s = lax.axis_index("s")

        def zero_deg(i, _):
            deg_v[pl.ds(i * 16, 16)] = jnp.zeros((16,), jnp.float32)
            return 0
        lax.fori_loop(0, NPAD // 16, zero_deg, 0)

        ztile = NPAD // NS
        pltpu.sync_copy(deg_v.at[pl.ds(0, ztile)],
                        deg_s.at[pl.ds(s * ztile, ztile)])
        plsc.subcore_barrier()

        pltpu.sync_copy(ew_h.at[pl.ds(s * ET, ET)], wall_v)
        ri_b = (ri0_v, ri1_v)
        isems = (i0, i1)
        ssems = (s0, s1)
        pltpu.async_copy(row_h.at[pl.ds(s * ET, B)], ri_b[0], isems[0])

        def deg_pair(i2, _):
            for b in range(2):
                bnx = (b + 1) % 2
                i = i2 * 2 + b

                @pl.when(i >= 1)
                def _():
                    pltpu.make_async_copy(
                        wall_v.at[pl.ds(i * B, B)],
                        deg_s.at[ri_b[bnx]], ssems[bnx]).wait()

                @pl.when(i + 1 < NBATCH)
                def _():
                    pltpu.async_copy(
                        row_h.at[pl.ds(s * ET + (i + 1) * B, B)],
                        ri_b[bnx], isems[bnx])

                pltpu.make_async_copy(
                    row_h.at[pl.ds(s * ET + i * B, B)],
                    ri_b[b], isems[b]).wait()
                pltpu.async_copy(wall_v.at[pl.ds(i * B, B)],
                                 deg_s.at[ri_b[b]], ssems[b], add=True)
            return 0
        lax.fori_loop(0, NBATCH // 2, deg_pair, 0)
        pltpu.make_async_copy(wall_v.at[pl.ds(0, B)],
                              deg_s.at[ri_b[1]], ssems[1]).wait()
        plsc.subcore_barrier()
        pltpu.sync_copy(deg_s, deg_v)

        def mkdis(i, _):
            sl = pl.ds(i * 16, 16)
            dis_v[sl] = _rsqrt16(deg_v[sl])
            return 0
        lax.fori_loop(0, NPAD // 16, mkdis, 0)

        wid = c * NS + s

        def lw_chunk(ch, _):
            base = wid * EWK + ch * CE
            pltpu.sync_copy(row_h.at[pl.ds(base, CE)], ri_v)
            pltpu.sync_copy(col_h.at[pl.ds(base, CE)], ci_v)
            pltpu.sync_copy(ew_h.at[pl.ds(base, CE)], w_v)

            def inner(q, _):
                sl = pl.ds(q * 16, 16)
                rv = ri_v[sl]
                cv = ci_v[sl]
                wv = w_v[sl]
                dr = plsc.load_gather(dis_v, [rv])
                dc = plsc.load_gather(dis_v, [cv])
                val = -(dr * wv * dc)
                lwo_v[sl] = jnp.where(rv == cv, val - 1.0, val)
                return 0
            lax.fori_loop(0, CE // 16, inner, 0)
            pltpu.sync_copy(lwo_v, lw_h.at[pl.ds(base, CE)])
            return 0
        lax.fori_loop(0, EWK // CE, lw_chunk, 0)

    return pre


def _make_prop(Q):
    mesh = plsc.VectorSubcoreMesh(core_axis_name="c", subcore_axis_name="s")
    tsd = jax.ShapeDtypeStruct((Q * NPAD, CH), jnp.float32)

    @functools.partial(
        pl.kernel,
        out_type=[tsd, tsd, tsd, tsd],
        mesh=mesh,
        compiler_params=pltpu.CompilerParams(needs_layout_passes=False,
                                             use_tc_tiling_on_sc=False),
        scratch_types=[
            pltpu.VMEM_SHARED((NPAD, CH), jnp.float32),
            pltpu.VMEM((ETM,), jnp.int32),
            pltpu.VMEM((BB,), jnp.int32),
            pltpu.VMEM((BB,), jnp.int32),
            pltpu.VMEM((BB,), jnp.int32),
            pltpu.VMEM((BB,), jnp.int32),
            pltpu.VMEM((BB,), jnp.float32),
            pltpu.VMEM((BB,), jnp.float32),
            pltpu.VMEM((BB,), jnp.float32),
            pltpu.VMEM((BB,), jnp.float32),
            pltpu.VMEM((BB, CH), jnp.float32),
            pltpu.VMEM((BB, CH), jnp.float32),
            pltpu.VMEM((BB, CH), jnp.float32),
            pltpu.VMEM((BB, CH), jnp.float32),
            pltpu.VMEM((RC, CH), jnp.float32),
            pltpu.VMEM((RC, CH), jnp.float32),
            pltpu.VMEM((TAIL,), jnp.int32),
            pltpu.VMEM((TAIL,), jnp.int32),
            pltpu.VMEM((TAIL,), jnp.float32),
            pltpu.VMEM((TAIL, CH), jnp.float32),
            pltpu.SemaphoreType.DMA,
            pltpu.SemaphoreType.DMA,
            pltpu.SemaphoreType.DMA,
            pltpu.SemaphoreType.DMA,
            pltpu.SemaphoreType.DMA,
            pltpu.SemaphoreType.DMA,
            pltpu.SemaphoreType.DMA,
            pltpu.SemaphoreType.DMA,
            pltpu.SemaphoreType.DMA,
            pltpu.SemaphoreType.DMA,
            pltpu.SemaphoreType.DMA,
            pltpu.SemaphoreType.DMA,
            pltpu.SemaphoreType.DMA,
            pltpu.SemaphoreType.DMA,
            pltpu.SemaphoreType.DMA,
            pltpu.SemaphoreType.DMA,
            pltpu.SemaphoreType.DMA,
        ],
    )
    def prop(t0_h, row_h, col_h, lw_h, t1_h, t2_h, t3_h, t4_h,
             acc_s, srca_v, d0_v, d1_v, d2_v, d3_v, w0_v, w1_v, w2_v, w3_v,
             r0_v, r1_v, r2_v, r3_v, p_v, tb_v,
             srct_v, dstt_v, lwt_v, rowst_v,
             g0, g1, g2, g3, s0, s1, s2, s3,
             e0, e1, e2, e3, f0, f1, f2, f3, st):
        rows_b = (r0_v, r1_v, r2_v, r3_v)
        dst_b = (d0_v, d1_v, d2_v, d3_v)
        lw_b = (w0_v, w1_v, w2_v, w3_v)
        gsems = (g0, g1, g2, g3)
        ssems = (s0, s1, s2, s3)
        dsems = (e0, e1, e2, e3)
        lsems = (f0, f1, f2, f3)
        c = lax.axis_index("c")
        s = lax.axis_index("s")
        rbase = s * RT
        mbase = s * ETM
        tbase = TBASE + s * TAIL
        QC = Q // 2

        pltpu.sync_copy(row_h.at[pl.ds(mbase, ETM)], srca_v)
        pltpu.sync_copy(row_h.at[pl.ds(tbase, TAIL)], srct_v)
        pltpu.sync_copy(lw_h.at[pl.ds(tbase, TAIL)], lwt_v)
        pltpu.sync_copy(col_h.at[pl.ds(tbase, TAIL)], dstt_v)

        def zrow(r, _):
            for v in range(CH // 16):
                tb_v[r, pl.ds(v * 16, 16)] = jnp.zeros((16,), jnp.float32)
            return 0
        lax.fori_loop(0, RC, zrow, 0)

        ins = [t0_h, t1_h, t2_h, t3_h]
        prevs = [None, t0_h, t1_h, t2_h]
        outs = [t1_h, t2_h, t3_h, t4_h]
        for q in range(QC):
            coff = (c * QC + q) * NPAD
            delta = coff if q == 0 else NPAD

            def adj(i, _):
                sl = pl.ds(i * 16, 16)
                srca_v[sl] = srca_v[sl] + delta
                return 0
            lax.fori_loop(0, ETM // 16, adj, 0, unroll=8)

            def adjt(i, _):
                sl = pl.ds(i * 16, 16)
                srct_v[sl] = srct_v[sl] + delta
                return 0
            lax.fori_loop(0, TAIL // 16, adjt, 0)

            for k in range(4):
                tin, tprev, tout = ins[k], prevs[k], outs[k]

                for j in range(RT // RC):
                    pltpu.sync_copy(tb_v, acc_s.at[pl.ds(rbase + j * RC, RC)])
                plsc.subcore_barrier()

                for pi in range(2):
                    pltpu.async_copy(
                        tin.at[srca_v.at[pl.ds(pi * BB, BB)]],
                        rows_b[pi], gsems[pi])
                    pltpu.async_copy(col_h.at[pl.ds(mbase + pi * BB, BB)],
                                     dst_b[pi], dsems[pi])
                    pltpu.async_copy(lw_h.at[pl.ds(mbase + pi * BB, BB)],
                                     lw_b[pi], lsems[pi])

                def quad(i4, _):
                    for b in range(4):
                        bnx = (b + 2) % 4
                        i = i4 * 4 + b

                        @pl.when(i >= 2)
                        def _():
                            pltpu.make_async_copy(
                                rows_b[bnx], acc_s.at[dst_b[bnx]],
                                ssems[bnx]).wait()

                        @pl.when(i + 2 < NB2)
                        def _():
                            pltpu.async_copy(
                                tin.at[srca_v.at[pl.ds((i + 2) * BB, BB)]],
                                rows_b[bnx], gsems[bnx])
                            pltpu.async_copy(
                                col_h.at[pl.ds(mbase + (i + 2) * BB, BB)],
                                dst_b[bnx], dsems[bnx])
                            pltpu.async_copy(
                                lw_h.at[pl.ds(mbase + (i + 2) * BB, BB)],
                                lw_b[bnx], lsems[bnx])

                        pltpu.make_async_copy(
                            tin.at[srca_v.at[pl.ds(i * BB, BB)]],
                            rows_b[b], gsems[b]).wait()
                        pltpu.make_async_copy(
                            lw_h.at[pl.ds(mbase + i * BB, BB)],
                            lw_b[b], lsems[b]).wait()
                        pltpu.make_async_copy(
                            col_h.at[pl.ds(mbase + i * BB, BB)],
                            dst_b[b], dsems[b]).wait()
                        rv = rows_b[b]
                        wv_ref = lw_b[b]

                        @plsc.parallel_loop(0, BB, unroll=4)
                        def _(e):
                            ev = jnp.zeros((16,), jnp.int32) + e
                            w = plsc.load_gather(wv_ref, [ev])
                            for v in range(CH // 16):
                                sl = pl.ds(v * 16, 16)
                                rv[e, sl] = rv[e, sl] * w

                        pltpu.async_copy(rv, acc_s.at[dst_b[b]],
                                         ssems[b], add=True)
                    return 0
                lax.fori_loop(0, NB2 // 4, quad, 0)
                pltpu.make_async_copy(rows_b[2], acc_s.at[dst_b[2]],
                                      ssems[2]).wait()
                pltpu.make_async_copy(rows_b[3], acc_s.at[dst_b[3]],
                                      ssems[3]).wait()

                pltpu.async_copy(tin.at[srct_v], rowst_v, st).wait()

                def tscale(e, ev):
                    w = plsc.load_gather(lwt_v, [ev])
                    for v in range(CH // 16):
                        sl = pl.ds(v * 16, 16)
                        rowst_v[e, sl] = rowst_v[e, sl] * w
                    return ev + 1
                lax.fori_loop(0, TAIL, tscale, jnp.zeros((16,), jnp.int32),
                              unroll=4)
                pltpu.sync_copy(rowst_v, acc_s.at[dstt_v], add=True)
                plsc.subcore_barrier()

                for j in range(RT // RC):
                    r0 = rbase + j * RC
                    pltpu.sync_copy(acc_s.at[pl.ds(r0, RC)], p_v)
                    if k == 0:
                        pltpu.sync_copy(p_v, tout.at[pl.ds(coff + r0, RC)])
                    else:
                        pltpu.sync_copy(tprev.at[pl.ds(coff + r0, RC)], tb_v)

                        def comb(r, _):
                            for v in range(CH // 16):
                                sl = pl.ds(v * 16, 16)
                                p_v[r, sl] = 2.0 * p_v[r, sl] - tb_v[r, sl]
                            return 0
                        lax.fori_loop(0, RC, comb, 0, unroll=2)
                        pltpu.sync_copy(p_v, tout.at[pl.ds(coff + r0, RC)])
                if q == QC - 1 and k == 3:
                    break
                plsc.subcore_barrier()
                if k >= 1:
                    lax.fori_loop(0, RC, zrow, 0)

    return prop


def _mm_body(bn, Qin, Hout, last, *refs):
    t_refs = refs[:KCH]
    w_ref, b_ref = refs[KCH], refs[KCH + 1]
    if last:
        wlin_ref, blin_ref = refs[KCH + 2], refs[KCH + 3]
        out_ref = refs[KCH + 4]
    else:
        out_ref = refs[KCH + 2]
    acc = jnp.zeros((bn, 2 * Hout), jnp.float32)
    for k in range(KCH):
        for h in range(Qin):
            acc = acc + jnp.dot(t_refs[k][h], w_ref[k, h],
                                preferred_element_type=jnp.float32)
    b = b_ref[0]
    z = jax.nn.sigmoid(acc[:, :Hout] + b[:Hout])
    t = jnp.tanh(acc[:, Hout:] + b[Hout:])
    hval = jax.nn.relu((1.0 - z) * t)
    if last:
        y = jnp.dot(hval, wlin_ref[...], preferred_element_type=jnp.float32)
        out_ref[...] = y + blin_ref[0, 0]
    else:
        Qout = out_ref.shape[0]
        w = Hout // Qout
        for qo in range(Qout):
            out_ref[qo] = hval[:, qo * w:(qo + 1) * w]


def _run_mm(ts, wc, bc, Qin, Hout, Qout=2, last=False, wlin=None, blin=None):
    bn = 1024
    grid = (NPAD // bn,)
    t_spec = pl.BlockSpec((Qin, bn, CH), lambda i: (0, i, 0))
    in_specs = [t_spec] * KCH + [
        pl.BlockSpec((KCH, Qin, CH, 2 * Hout), lambda i: (0, 0, 0, 0)),
        pl.BlockSpec((1, 2 * Hout), lambda i: (0, 0)),
    ]
    args = list(ts) + [wc, bc]
    if last:
        in_specs += [
            pl.BlockSpec((Hout, 1), lambda i: (0, 0)),
            pl.BlockSpec((1, 1), lambda i: (0, 0)),
        ]
        args += [wlin, blin]
        out_spec = pl.BlockSpec((bn, 1), lambda i: (i, 0))
        out_shape = jax.ShapeDtypeStruct((NPAD, 1), jnp.float32)
    else:
        out_spec = pl.BlockSpec((Qout, bn, Hout // Qout), lambda i: (0, i, 0))
        out_shape = jax.ShapeDtypeStruct((Qout, NPAD, Hout // Qout),
                                         jnp.float32)
    fn = pl.pallas_call(
        functools.partial(_mm_body, bn, Qin, Hout, last),
        grid=grid,
        in_specs=in_specs,
        out_specs=out_spec,
        out_shape=out_shape,
    )
    return fn(*args)


def _mkw(Wx, bx, bh, Qin):
    K, ic, oc = Wx.shape[1], Wx.shape[2], Wx.shape[3]
    wc = jnp.concatenate([Wx[0], Wx[2]], axis=-1)
    wc = wc.reshape(K, Qin, ic // Qin, 2 * oc)
    bc = jnp.concatenate([bx[0] + bh[0], bx[2] + bh[2]])
    return wc, bc.reshape(1, 2 * oc)


def kernel(x, edge_index, edge_weight, Wx1, Wh1, bx1, bh1, Wx2, Wh2, bx2, bh2,
           Wx3, Wh3, bx3, bh3, Wlin, blin):
    row = edge_index[0]
    col = edge_index[1]
    pre = _make_pre()
    lw = pre(row, col, edge_weight)

    prop2 = _make_prop(2)
    prop4 = _make_prop(4)

    def layer(xinq, Qin, Wx, bx, bh, Hout, Qout, last=False):
        propfn = prop2 if Qin == 2 else prop4
        t1, t2, t3, t4 = propfn(xinq, row, col, lw)
        wc, bc = _mkw(Wx, bx, bh, Qin)
        ts = [a.reshape(Qin, NPAD, CH) for a in (xinq, t1, t2, t3, t4)]
        if last:
            return _run_mm(ts, wc, bc, Qin, Hout, last=True,
                           wlin=Wlin, blin=blin.reshape(1, 1))[:N, 0]
        h = _run_mm(ts, wc, bc, Qin, Hout, Qout=Qout)
        return h.reshape(Qout * NPAD, CH)

    pad = ((0, NPAD - N), (0, 0))
    x2 = jnp.concatenate(
        [jnp.pad(x[:, :64], pad), jnp.pad(x[:, 64:], pad)], axis=0)
    h1 = layer(x2, 2, Wx1, bx1, bh1, 256, 4)
    h2 = layer(h1, 4, Wx2, bx2, bh2, 128, 2)
    y = layer(h2, 2, Wx3, bx3, bh3, 64, 1, last=True)
    return y
